# baseline (speedup 1.0000x reference)
import math

import jax
import jax.numpy as jnp
from jax import lax
from jax.experimental import pallas as pl
from jax.experimental.pallas import tpu as pltpu
from jax.experimental.pallas import tpu_sc as plsc

D_LANES = 16
NUM_CORES = 2
NUM_SUBCORES = 16
NW = NUM_CORES * NUM_SUBCORES


def _make_gather(B: int, V: int, D: int, chunk: int):
    assert B % NW == 0
    bpw = B // NW
    assert bpw % chunk == 0
    nchunk = bpw // chunk
    assert nchunk >= 2
    assert chunk % 8 == 0
    assert D % D_LANES == 0
    scale = math.sqrt(float(D))
    vregs_per_row = D // D_LANES

    mesh = plsc.VectorSubcoreMesh(core_axis_name="c", subcore_axis_name="s")

    @pl.kernel(
        out_type=jax.ShapeDtypeStruct((B, D), jnp.float32),
        mesh=mesh,
        compiler_params=pltpu.CompilerParams(use_tc_tiling_on_sc=True),
        scratch_types=[
            pltpu.VMEM((bpw,), jnp.int32),
            pltpu.VMEM((chunk, D), jnp.float32),
            pltpu.VMEM((chunk, D), jnp.float32),
            pltpu.SemaphoreType.DMA,
            pltpu.SemaphoreType.DMA,
            pltpu.SemaphoreType.DMA,
            pltpu.SemaphoreType.DMA,
        ],
    )
    def gather_scaled(tokens_hbm, table_hbm, out_hbm,
                      idx_v, buf0, buf1, sg0, sg1, so0, so1):
        wid = lax.axis_index("s") * NUM_CORES + lax.axis_index("c")
        base = wid * bpw
        pltpu.sync_copy(tokens_hbm.at[pl.ds(base, bpw)], idx_v)

        bufs = (buf0, buf1)
        sgs = (sg0, sg1)
        sos = (so0, so1)

        def gather_start(g):
            b = g % 2
            return pltpu.async_copy(
                table_hbm.at[idx_v.at[pl.ds(g * chunk, chunk)]], bufs[b], sgs[b]
            )

        gh = [None] * nchunk
        oh = [None] * nchunk
        gh[0] = gather_start(0)
        for g in range(nchunk):
            b = g % 2
            if g + 1 < nchunk:
                if g >= 1:
                    oh[g - 1].wait()
                gh[g + 1] = gather_start(g + 1)
            gh[g].wait()

            buf = bufs[b]

            @plsc.parallel_loop(0, chunk, 1, unroll=2)
            def _(r):
                for d in range(vregs_per_row):
                    sl = pl.ds(d * D_LANES, D_LANES)
                    buf[r, sl] = buf[r, sl] * scale

            oh[g] = pltpu.async_copy(
                buf, out_hbm.at[pl.ds(base + g * chunk, chunk)], sos[b]
            )
        oh[nchunk - 2].wait()
        oh[nchunk - 1].wait()

    return gather_scaled


def kernel(tokens, table):
    assert tokens.ndim == 2
    V, D = table.shape
    B = tokens.shape[0] * tokens.shape[1]
    flat = tokens.reshape(B).astype(jnp.int32)
    gather = _make_gather(B, V, D, chunk=400)
    out = gather(flat, table)
    return out.reshape(tokens.shape[0], tokens.shape[1], D)


# speedup vs baseline: 3.0070x; 3.0070x over previous
import math

import jax
import jax.numpy as jnp
from jax import lax
from jax.experimental import pallas as pl
from jax.experimental.pallas import tpu as pltpu
from jax.experimental.pallas import tpu_sc as plsc

D_LANES = 16
NUM_CORES = 2
NUM_SUBCORES = 16
NW = NUM_CORES * NUM_SUBCORES


def _make_gather(B: int, V: int, D: int, chunk: int):
    assert B % NW == 0
    bpw = B // NW
    assert bpw % chunk == 0
    nchunk = bpw // chunk
    assert nchunk >= 2
    assert chunk % 8 == 0
    assert D % D_LANES == 0
    scale = math.sqrt(float(D))
    vregs_per_row = D // D_LANES

    mesh = plsc.VectorSubcoreMesh(core_axis_name="c", subcore_axis_name="s")

    @pl.kernel(
        out_type=jax.ShapeDtypeStruct((B, D), jnp.float32),
        mesh=mesh,
        compiler_params=pltpu.CompilerParams(use_tc_tiling_on_sc=True),
        scratch_types=[
            pltpu.VMEM((bpw,), jnp.int32),
            pltpu.VMEM((chunk, D), jnp.float32),
            pltpu.VMEM((chunk, D), jnp.float32),
            pltpu.SemaphoreType.DMA,
            pltpu.SemaphoreType.DMA,
            pltpu.SemaphoreType.DMA,
            pltpu.SemaphoreType.DMA,
        ],
    )
    def gather_scaled(tokens_hbm, table_hbm, out_hbm,
                      idx_v, buf0, buf1, sg0, sg1, so0, so1):
        wid = lax.axis_index("s") * NUM_CORES + lax.axis_index("c")
        base = wid * bpw
        pltpu.sync_copy(tokens_hbm.at[pl.ds(base, bpw)], idx_v)

        bufs = (buf0, buf1)
        sgs = (sg0, sg1)
        sos = (so0, so1)

        def gather_start(g):
            b = g % 2
            return pltpu.async_copy(
                table_hbm.at[idx_v.at[pl.ds(g * chunk, chunk)]], bufs[b], sgs[b]
            )

        gh = [None] * nchunk
        oh = [None] * nchunk
        gh[0] = gather_start(0)
        for g in range(nchunk):
            b = g % 2
            if g + 1 < nchunk:
                if g >= 1:
                    oh[g - 1].wait()
                gh[g + 1] = gather_start(g + 1)
            gh[g].wait()

            buf = bufs[b]

            @plsc.parallel_loop(0, chunk, 1, unroll=2)
            def _(r):
                for d in range(vregs_per_row):
                    sl = pl.ds(d * D_LANES, D_LANES)
                    buf[r, sl] = buf[r, sl] * scale

            oh[g] = pltpu.async_copy(
                buf, out_hbm.at[pl.ds(base + g * chunk, chunk)], sos[b]
            )
        oh[nchunk - 2].wait()
        oh[nchunk - 1].wait()

    return gather_scaled


def kernel(tokens, table):
    assert tokens.ndim == 2
    V, D = table.shape
    S, W = tokens.shape
    B = S * W
    flat = tokens.T.reshape(B).astype(jnp.int32)
    gather = _make_gather(B, V, D, chunk=400)
    out = gather(flat, table)
    return out.reshape(W, S, D).transpose(1, 0, 2)
